# Initial kernel scaffold; baseline (speedup 1.0000x reference)
import jax
import jax.numpy as jnp
from jax.experimental import pallas as pl


def kernel(indices, time_steps, table, time_table, gamma, beta):
    raise NotImplementedError("write your pallas kernel here")


# speedup vs baseline: 4.7141x; 4.7141x over previous
import functools

import jax
import jax.numpy as jnp
from jax import lax
from jax.experimental import pallas as pl
from jax.experimental.pallas import tpu as pltpu
from jax.experimental.pallas import tpu_sc as plsc

DIM = 64
LANES = 16
NJ = DIM // LANES
NC = 2
NS = 16
NW = NC * NS
CHUNK = 128
UNROLL = 4


def _rsqrt16(x):
    i = plsc.bitcast(x, jnp.int32)
    i = jnp.int32(0x5F3759DF) - (i >> 1)
    y = plsc.bitcast(i, jnp.float32)
    half = x * jnp.float32(0.5)
    for _ in range(3):
        y = y * (jnp.float32(1.5) - half * y * y)
    return y


def _ln_chunk(e_ref, t_ref, o_ref, gvecs, bvecs):
    def body(it, carry):
        base = it * UNROLL
        for u in range(UNROLL):
            r = base + u
            h = [e_ref[r, pl.ds(16 * j, 16)] + t_ref[r, pl.ds(16 * j, 16)]
                 for j in range(NJ)]
            s = (h[0] + h[1]) + (h[2] + h[3])
            ss = (h[0] * h[0] + h[1] * h[1]) + (h[2] * h[2] + h[3] * h[3])
            tot = jnp.sum(s)
            sstot = jnp.sum(ss)
            mu = tot * jnp.float32(1.0 / DIM)
            var = sstot * jnp.float32(1.0 / DIM) - mu * mu
            xv = jnp.full((16,), var + jnp.float32(1e-5), jnp.float32)
            rstd = _rsqrt16(xv)
            muv = jnp.full((16,), mu, jnp.float32)
            for j in range(NJ):
                o_ref[r, pl.ds(16 * j, 16)] = (h[j] - muv) * rstd * gvecs[j] + bvecs[j]
        return carry

    lax.fori_loop(0, CHUNK // UNROLL, body, 0, unroll=False)


def _sc_body(idx_hbm, ts_hbm, table_hbm, ttable_hbm, gamma_hbm, beta_hbm,
             out_hbm, idx_v, ts_v, e0, e1, t0, t1, o0, o1, gam_v, bet_v,
             se0, se1, st0, st1, so0, so1, nchunk):
    wid = lax.axis_index("s") * NC + lax.axis_index("c")
    rows_per_w = nchunk * CHUNK
    wbase = wid * rows_per_w

    pltpu.sync_copy(idx_hbm.at[wid], idx_v)
    pltpu.sync_copy(ts_hbm.at[wid], ts_v)
    pltpu.sync_copy(gamma_hbm, gam_v)
    pltpu.sync_copy(beta_hbm, bet_v)
    gvecs = [gam_v[pl.ds(16 * j, 16)] for j in range(NJ)]
    bvecs = [bet_v[pl.ds(16 * j, 16)] for j in range(NJ)]

    ebufs, tbufs, obufs = (e0, e1), (t0, t1), (o0, o1)
    esems, tsems, osems = (se0, se1), (st0, st1), (so0, so1)

    def issue_gather(i, p):
        pltpu.async_copy(table_hbm.at[idx_v.at[i]], ebufs[p], esems[p])
        pltpu.async_copy(ttable_hbm.at[ts_v.at[i]], tbufs[p], tsems[p])

    def wait_gather(i, p):
        pltpu.make_async_copy(table_hbm.at[idx_v.at[i]], ebufs[p], esems[p]).wait()
        pltpu.make_async_copy(ttable_hbm.at[ts_v.at[i]], tbufs[p], tsems[p]).wait()

    def out_slice(i):
        return out_hbm.at[pl.ds(wbase + i * CHUNK, CHUNK)]

    def step(i, p, issue_next, wait_prev_scatter):
        wait_gather(i, p)
        if issue_next:
            issue_gather(i + 1, 1 - p)
        if wait_prev_scatter:
            pltpu.make_async_copy(obufs[p], out_slice(i - 2), osems[p]).wait()
        _ln_chunk(ebufs[p], tbufs[p], obufs[p], gvecs, bvecs)
        pltpu.async_copy(obufs[p], out_slice(i), osems[p])

    issue_gather(0, 0)
    step(0, 0, True, False)
    step(1, 1, True, False)

    def loop_body(g, carry):
        i = 2 * g
        step(i, 0, True, True)
        step(i + 1, 1, True, True)
        return carry

    lax.fori_loop(1, (nchunk - 2) // 2, loop_body, 0, unroll=False)

    step(nchunk - 2, 0, True, True)
    step(nchunk - 1, 1, False, True)
    pltpu.make_async_copy(obufs[0], out_slice(nchunk - 2), osems[0]).wait()
    pltpu.make_async_copy(obufs[1], out_slice(nchunk - 1), osems[1]).wait()


@functools.partial(jax.jit, static_argnames=("n", "nchunk"))
def _run(idx, ts, table, time_table, gamma, beta, n, nchunk):
    mesh = plsc.VectorSubcoreMesh(core_axis_name="c", subcore_axis_name="s",
                                  num_cores=NC, num_subcores=NS)
    body = functools.partial(_sc_body, nchunk=nchunk)
    f = pl.kernel(
        body,
        out_type=jax.ShapeDtypeStruct((n, DIM), jnp.float32),
        mesh=mesh,
        compiler_params=pltpu.CompilerParams(
            needs_layout_passes=False, use_tc_tiling_on_sc=False),
        scratch_types=[
            pltpu.VMEM((nchunk, CHUNK), jnp.int32),
            pltpu.VMEM((nchunk, CHUNK), jnp.int32),
            pltpu.VMEM((CHUNK, DIM), jnp.float32),
            pltpu.VMEM((CHUNK, DIM), jnp.float32),
            pltpu.VMEM((CHUNK, DIM), jnp.float32),
            pltpu.VMEM((CHUNK, DIM), jnp.float32),
            pltpu.VMEM((CHUNK, DIM), jnp.float32),
            pltpu.VMEM((CHUNK, DIM), jnp.float32),
            pltpu.VMEM((DIM,), jnp.float32),
            pltpu.VMEM((DIM,), jnp.float32),
            pltpu.SemaphoreType.DMA,
            pltpu.SemaphoreType.DMA,
            pltpu.SemaphoreType.DMA,
            pltpu.SemaphoreType.DMA,
            pltpu.SemaphoreType.DMA,
            pltpu.SemaphoreType.DMA,
        ],
    )
    return f(idx, ts, table, time_table, gamma, beta)


def kernel(indices, time_steps, table, time_table, gamma, beta):
    b, l = indices.shape
    n = b * l
    assert n % (NW * CHUNK) == 0
    nchunk = n // (NW * CHUNK)
    assert nchunk % 2 == 0
    idx = indices.reshape(NW, nchunk, CHUNK).astype(jnp.int32)
    ts = time_steps.reshape(NW, nchunk, CHUNK).astype(jnp.int32)
    out = _run(idx, ts, table.astype(jnp.float32), time_table.astype(jnp.float32),
               gamma.astype(jnp.float32), beta.astype(jnp.float32), n, nchunk)
    return out.reshape(b, l, DIM)
